# baseline (speedup 1.0000x reference)
import functools

import jax
import jax.numpy as jnp
from jax import lax
from jax.experimental import pallas as pl
from jax.experimental.pallas import tpu as pltpu
from jax.experimental.pallas import tpu_sc as plsc

OUT_H = 7
OUT_W = 7
SCALE = 0.0625
NEG = -3.4e38
R7 = 0.14285714924335479736328125

N_IMG = 2
C = 256
H = 50
W = 50
K = 1000
REG = 12
NCHUNK = C // 16
RPT = 32
ROWSZ = REG * C
OUTSZ = C * OUT_H * OUT_W


def _roipool_sc_body(tbl_hbm, rois_hbm, out_hbm,
                     reg_a, reg_b, outbuf_a, outbuf_b, rois_v,
                     sem_a, sem_b, osem_a, osem_b):
  nc = 2
  wid = lax.axis_index("s") * nc + lax.axis_index("c")
  start = jnp.minimum(RPT * wid, K - RPT)
  pltpu.sync_copy(rois_hbm.at[pl.ds(start * 16, RPT * 16)], rois_v)
  lane49 = lax.iota(jnp.int32, 16) * (OUT_H * OUT_W)

  def fl7(v):
    q = v.astype(jnp.float32) * jnp.float32(R7)
    t = q.astype(jnp.int32)
    return t - (t.astype(jnp.float32) > q).astype(jnp.int32)

  def ce7(v):
    q = v.astype(jnp.float32) * jnp.float32(R7)
    t = q.astype(jnp.int32)
    return t + (t.astype(jnp.float32) < q).astype(jnp.int32)

  def geo(r):
    vec = rois_v[pl.ds(r * 16, 16)]
    b = vec[0].astype(jnp.int32)
    x1 = (vec[1] * SCALE).astype(jnp.int32)
    y1 = (vec[2] * SCALE).astype(jnp.int32)
    x2 = (vec[3] * SCALE).astype(jnp.int32)
    y2 = (vec[4] * SCALE).astype(jnp.int32)
    rw = jnp.maximum(x2 - x1 + 1, 1)
    rh = jnp.maximum(y2 - y1 + 1, 1)
    yoff = jnp.minimum(y1, H - REG)
    xoff = jnp.minimum(x1, W - REG)
    return b, x1, y1, rw, rh, yoff, xoff

  def fire_region(reg, sem, r):
    b, _, _, _, _, yoff, xoff = geo(r)
    pltpu.async_copy(
        tbl_hbm.at[pl.ds(b * H + yoff, REG), :, pl.ds(xoff * C, ROWSZ)],
        reg, sem)

  def drain_region(reg, sem):
    pltpu.make_async_copy(
        tbl_hbm.at[pl.ds(0, REG), :, pl.ds(0, ROWSZ)], reg, sem).wait()

  def drain_out(outbuf, osem):
    pltpu.make_async_copy(
        out_hbm.at[:, :, pl.ds(0, C)], outbuf, osem).wait()

  def compute_roi_wyx(reg, outbuf, r, wy, wx):
    _, x1, y1, rw, rh, _, _ = geo(r)
    yoff = jnp.minimum(y1, H - REG)
    xoff = jnp.minimum(x1, W - REG)

    def do_i(i, carry):
      hs = jnp.minimum(fl7(i * rh) + y1, H)
      he = jnp.minimum(ce7((i + 1) * rh) + y1, H)
      cy = he - hs
      cym = jnp.maximum(cy - 1, 0)
      ry0 = hs - yoff
      rows = [ry0]
      if wy == 3:
        rows.append(ry0 + jnp.minimum(1, cym))
      rows.append(ry0 + cym)

      def do_j(j, carry):
        ws = jnp.minimum(fl7(j * rw) + x1, W)
        we = jnp.minimum(ce7((j + 1) * rw) + x1, W)
        cx = we - ws
        cxm = jnp.maximum(cx - 1, 0)
        rx0 = ws - xoff
        cols = [rx0 * C]
        if wx == 3:
          cols.append((rx0 + jnp.minimum(1, cxm)) * C)
        cols.append((rx0 + cxm) * C)
        bases = [(rr, cc) for rr in rows for cc in cols]
        valid = (cy > 0) & (cx > 0)
        cell = i * OUT_W + j
        for ch in range(NCHUNK):
          off = ch * 16
          vs = [reg[bs[0], 0, pl.ds(bs[1] + off, 16)] for bs in bases]
          while len(vs) > 1:
            vs = ([jnp.maximum(vs[2 * t], vs[2 * t + 1])
                   for t in range(len(vs) // 2)]
                  + ([vs[-1]] if len(vs) % 2 else []))
          outbuf[cell, 0, pl.ds(off, 16)] = jnp.where(valid, vs[0],
                                                      jnp.float32(0.0))
        return carry

      return lax.fori_loop(0, OUT_W, do_j, carry)

    lax.fori_loop(0, OUT_H, do_i, 0)

  def compute_roi(reg, outbuf, r):
    _, _, _, rw, rh, _, _ = geo(r)
    ysmall = rh <= OUT_H
    xsmall = rw <= OUT_W
    for wy, wx, cond in [(2, 2, ysmall & xsmall),
                         (2, 3, ysmall & (~xsmall)),
                         (3, 2, (~ysmall) & xsmall),
                         (3, 3, (~ysmall) & (~xsmall))]:
      @pl.when(cond)
      def _(wy=wy, wx=wx):
        compute_roi_wyx(reg, outbuf, r, wy, wx)

  fire_region(reg_a, sem_a, 0)
  fire_region(reg_b, sem_b, 1)

  def do_pair(p, carry):
    for half, (reg, sem, outbuf, osem) in enumerate(
        [(reg_a, sem_a, outbuf_a, osem_a),
         (reg_b, sem_b, outbuf_b, osem_b)]):
      r = 2 * p + half
      drain_region(reg, sem)

      @pl.when(p >= 1)
      def _():
        drain_out(outbuf, osem)

      compute_roi(reg, outbuf, r)
      k = start + r
      pltpu.async_copy(outbuf, out_hbm.at[:, :, pl.ds(k * C, C)], osem)

      @pl.when(p < 15)
      def _():
        fire_region(reg, sem, r + 2)
    return carry

  lax.fori_loop(0, RPT // 2, do_pair, 0)
  drain_out(outbuf_a, osem_a)
  drain_out(outbuf_b, osem_b)


@jax.jit
def kernel(features, rois):
  tbl = (jnp.transpose(features, (0, 2, 3, 1))
         .reshape(N_IMG * H, 1, W * C))
  rois16 = (jnp.zeros((K + 8, 16), jnp.float32).at[:K, :5].set(rois)
            .reshape((K + 8) * 16))
  mesh = plsc.VectorSubcoreMesh(core_axis_name="c", subcore_axis_name="s")
  fn = pl.kernel(
      _roipool_sc_body,
      out_type=jax.ShapeDtypeStruct((OUT_H * OUT_W, 1, K * C), jnp.float32),
      mesh=mesh,
      compiler_params=pltpu.CompilerParams(needs_layout_passes=False),
      scratch_types=[
          pltpu.VMEM((REG, 1, ROWSZ), jnp.float32),
          pltpu.VMEM((REG, 1, ROWSZ), jnp.float32),
          pltpu.VMEM((OUT_H * OUT_W, 1, C), jnp.float32),
          pltpu.VMEM((OUT_H * OUT_W, 1, C), jnp.float32),
          pltpu.VMEM((RPT * 16,), jnp.float32),
          pltpu.SemaphoreType.DMA,
          pltpu.SemaphoreType.DMA,
          pltpu.SemaphoreType.DMA,
          pltpu.SemaphoreType.DMA,
      ],
  )
  out = fn(tbl, rois16).reshape(OUT_H, OUT_W, K, C)
  return jnp.transpose(out, (2, 3, 0, 1))


# speedup vs baseline: 1.4083x; 1.4083x over previous
import functools

import jax
import jax.numpy as jnp
from jax import lax
from jax.experimental import pallas as pl
from jax.experimental.pallas import tpu as pltpu
from jax.experimental.pallas import tpu_sc as plsc

OUT_H = 7
OUT_W = 7
SCALE = 0.0625
NEG = -3.4e38
R7 = 0.14285714924335479736328125

N_IMG = 2
C = 256
H = 50
W = 50
K = 1000
REG = 12
NCHUNK = C // 16
RPT = 32
ROWSZ = REG * C
OUTSZ = C * OUT_H * OUT_W


def _roipool_sc_body(tbl_hbm, rois_hbm, out_hbm,
                     reg_a, reg_b, outbuf_a, outbuf_b, rois_v,
                     sem_a, sem_b, osem_a, osem_b):
  nc = 2
  wid = lax.axis_index("s") * nc + lax.axis_index("c")
  start = jnp.minimum(RPT * wid, K - RPT)
  pltpu.sync_copy(rois_hbm.at[pl.ds(start * 16, RPT * 16)], rois_v)
  lane49 = lax.iota(jnp.int32, 16) * (OUT_H * OUT_W)

  def fl7(v):
    q = v.astype(jnp.float32) * jnp.float32(R7)
    t = q.astype(jnp.int32)
    return t - (t.astype(jnp.float32) > q).astype(jnp.int32)

  def ce7(v):
    q = v.astype(jnp.float32) * jnp.float32(R7)
    t = q.astype(jnp.int32)
    return t + (t.astype(jnp.float32) < q).astype(jnp.int32)

  def geo(r):
    vec = rois_v[pl.ds(r * 16, 16)]
    b = vec[0].astype(jnp.int32)
    x1 = (vec[1] * SCALE).astype(jnp.int32)
    y1 = (vec[2] * SCALE).astype(jnp.int32)
    x2 = (vec[3] * SCALE).astype(jnp.int32)
    y2 = (vec[4] * SCALE).astype(jnp.int32)
    rw = jnp.maximum(x2 - x1 + 1, 1)
    rh = jnp.maximum(y2 - y1 + 1, 1)
    yoff = jnp.minimum(y1, H - REG)
    xsmall = rw <= OUT_W
    xoff = jnp.where(xsmall, jnp.minimum(x1, W - 8), jnp.minimum(x1, W - REG))
    nrows = jnp.minimum(jnp.minimum(y1 + rh + 1, H) - yoff, REG)
    return b, x1, y1, rw, rh, yoff, xoff, xsmall, nrows

  def fire_region(reg, sem, r):
    b, _, _, _, _, yoff, xoff, xsmall, nrows = geo(r)
    base = (b * H + yoff) * W + xoff
    for width, cond in ((8 * C, xsmall), (ROWSZ, ~xsmall)):
      @pl.when(cond)
      def _(width=width):
        def fire(dy, carry):
          pltpu.async_copy(tbl_hbm.at[pl.ds((base + dy * W) * C, width)],
                           reg.at[pl.ds(dy * ROWSZ, width)], sem)
          return carry
        lax.fori_loop(0, nrows, fire, 0)

  def drain_region(reg, sem, r):
    _, _, _, _, _, _, _, xsmall, nrows = geo(r)
    for width, cond in ((8 * C, xsmall), (ROWSZ, ~xsmall)):
      @pl.when(cond)
      def _(width=width):
        def drain(dy, carry):
          pltpu.make_async_copy(tbl_hbm.at[pl.ds(0, width)],
                                reg.at[pl.ds(0, width)], sem).wait()
          return carry
        lax.fori_loop(0, nrows, drain, 0)

  def drain_out(outbuf, osem):
    pltpu.make_async_copy(out_hbm.at[pl.ds(0, OUTSZ)], outbuf, osem).wait()

  def compute_roi_wyx(reg, outbuf, r, wy, wx):
    _, x1, y1, rw, rh, yoff, xoff, _, _ = geo(r)

    def do_i(i, carry):
      hs = jnp.minimum(fl7(i * rh) + y1, H)
      he = jnp.minimum(ce7((i + 1) * rh) + y1, H)
      cy = he - hs
      cym = jnp.maximum(cy - 1, 0)
      ry0 = hs - yoff
      rows = [ry0 * ROWSZ]
      if wy == 3:
        rows.append((ry0 + jnp.minimum(1, cym)) * ROWSZ)
      rows.append((ry0 + cym) * ROWSZ)

      def do_j(j, carry):
        ws = jnp.minimum(fl7(j * rw) + x1, W)
        we = jnp.minimum(ce7((j + 1) * rw) + x1, W)
        cx = we - ws
        cxm = jnp.maximum(cx - 1, 0)
        rx0 = ws - xoff
        cols = [rx0 * C]
        if wx == 3:
          cols.append((rx0 + jnp.minimum(1, cxm)) * C)
        cols.append((rx0 + cxm) * C)
        bases = [rr + cc for rr in rows for cc in cols]
        valid = (cy > 0) & (cx > 0)
        cbase = (i * OUT_W + j) * C
        for ch in range(NCHUNK):
          off = ch * 16
          vs = [reg[pl.ds(bs + off, 16)] for bs in bases]
          while len(vs) > 1:
            vs = ([jnp.maximum(vs[2 * t], vs[2 * t + 1])
                   for t in range(len(vs) // 2)]
                  + ([vs[-1]] if len(vs) % 2 else []))
          outbuf[pl.ds(cbase + off, 16)] = jnp.where(valid, vs[0],
                                                     jnp.float32(0.0))
        return carry

      return lax.fori_loop(0, OUT_W, do_j, carry)

    lax.fori_loop(0, OUT_H, do_i, 0)

  def compute_roi(reg, outbuf, r):
    _, _, _, rw, rh, _, _, _, _ = geo(r)
    ysmall = rh <= OUT_H
    xsmall = rw <= OUT_W
    for wy, wx, cond in [(2, 2, ysmall & xsmall),
                         (2, 3, ysmall & (~xsmall)),
                         (3, 2, (~ysmall) & xsmall),
                         (3, 3, (~ysmall) & (~xsmall))]:
      @pl.when(cond)
      def _(wy=wy, wx=wx):
        compute_roi_wyx(reg, outbuf, r, wy, wx)

  fire_region(reg_a, sem_a, 0)
  fire_region(reg_b, sem_b, 1)

  def do_pair(p, carry):
    for half, (reg, sem, outbuf, osem) in enumerate(
        [(reg_a, sem_a, outbuf_a, osem_a),
         (reg_b, sem_b, outbuf_b, osem_b)]):
      r = 2 * p + half
      drain_region(reg, sem, r)

      @pl.when(p >= 1)
      def _():
        drain_out(outbuf, osem)

      compute_roi(reg, outbuf, r)
      k = start + r
      for cell in range(OUT_H * OUT_W):
        pltpu.async_copy(outbuf.at[pl.ds(cell * C, C)],
                         out_hbm.at[pl.ds((cell * K + k) * C, C)], osem)

      @pl.when(p < 15)
      def _():
        fire_region(reg, sem, r + 2)
    return carry

  lax.fori_loop(0, RPT // 2, do_pair, 0)
  drain_out(outbuf_a, osem_a)
  drain_out(outbuf_b, osem_b)


@jax.jit
def kernel(features, rois):
  tbl = jnp.transpose(features, (0, 2, 3, 1)).reshape(N_IMG * H * W * C)
  rois16 = (jnp.zeros((K + 8, 16), jnp.float32).at[:K, :5].set(rois)
            .reshape((K + 8) * 16))
  mesh = plsc.VectorSubcoreMesh(core_axis_name="c", subcore_axis_name="s")
  fn = pl.kernel(
      _roipool_sc_body,
      out_type=jax.ShapeDtypeStruct((K * OUTSZ,), jnp.float32),
      mesh=mesh,
      compiler_params=pltpu.CompilerParams(needs_layout_passes=False),
      scratch_types=[
          pltpu.VMEM((REG * ROWSZ,), jnp.float32),
          pltpu.VMEM((REG * ROWSZ,), jnp.float32),
          pltpu.VMEM((OUTSZ,), jnp.float32),
          pltpu.VMEM((OUTSZ,), jnp.float32),
          pltpu.VMEM((RPT * 16,), jnp.float32),
          pltpu.SemaphoreType.DMA,
          pltpu.SemaphoreType.DMA,
          pltpu.SemaphoreType.DMA,
          pltpu.SemaphoreType.DMA,
      ],
  )
  out = fn(tbl, rois16).reshape(OUT_H, OUT_W, K, C)
  return jnp.transpose(out, (2, 3, 0, 1))
